# Initial kernel scaffold; baseline (speedup 1.0000x reference)
import functools

import jax
import jax.numpy as jnp
from jax import lax
from jax.experimental import pallas as pl
from jax.experimental.pallas import tpu as pltpu
from jax.experimental.pallas import tpu_sc as plsc

_NUM_ROWS = 1000000
_EMBED = 16
_Q_MASK = jnp.int32(-65536)
_R_MASK = jnp.int32(65535)

_NC = 2
_NS = 16
_NW = _NC * _NS
_L = 16

_INV_ROWS = jnp.float32(1.0 / _NUM_ROWS)


def _mod_rows(x):
    q = (x.astype(jnp.float32) * _INV_ROWS).astype(jnp.int32)
    r = x - q * _NUM_ROWS
    r = jnp.where(r < 0, r + _NUM_ROWS, r)
    r = jnp.where(r >= _NUM_ROWS, r - _NUM_ROWS, r)
    return r


@functools.partial(jax.jit, static_argnames=("n",))
def _lookup_add(ids_flat, table, n):
    per_w = n // _NW
    chunk = 1024
    n_chunks = per_w // chunk
    n_vecs = per_w // _L

    mesh = plsc.VectorSubcoreMesh(core_axis_name="c", subcore_axis_name="s")

    @functools.partial(
        pl.kernel,
        out_type=jax.ShapeDtypeStruct((n, _EMBED), jnp.float32),
        mesh=mesh,
        scratch_types=[
            pltpu.VMEM((per_w,), jnp.int32),
            pltpu.VMEM((per_w,), jnp.int32),
            pltpu.VMEM((per_w,), jnp.int32),
            pltpu.VMEM((chunk, _EMBED), jnp.float32),
            pltpu.VMEM((chunk, _EMBED), jnp.float32),
            pltpu.SemaphoreType.DMA,
            pltpu.SemaphoreType.DMA,
        ],
    )
    def sc_kernel(ids_hbm, table_hbm, out_hbm,
                  ids_v, idxq_v, idxr_v, bufq, bufr, semq, semr):
        wid = lax.axis_index("s") * _NC + lax.axis_index("c")
        base = wid * per_w

        pltpu.sync_copy(ids_hbm.at[pl.ds(base, per_w)], ids_v)

        def idx_body(i, carry):
            o = i * _L
            x = ids_v[pl.ds(o, _L)]
            idxq_v[pl.ds(o, _L)] = _mod_rows(x & _Q_MASK)
            idxr_v[pl.ds(o, _L)] = x & _R_MASK
            return carry

        lax.fori_loop(0, n_vecs, idx_body, 0, unroll=4)

        def chunk_body(c, carry):
            co = c * chunk
            cpq = pltpu.async_copy(
                table_hbm.at[idxq_v.at[pl.ds(co, chunk)]], bufq, semq)
            cpr = pltpu.async_copy(
                table_hbm.at[idxr_v.at[pl.ds(co, chunk)]], bufr, semr)
            cpq.wait()
            cpr.wait()

            def add_body(i, carry2):
                bufq[i] = bufq[i] + bufr[i]
                return carry2

            lax.fori_loop(0, chunk, add_body, 0, unroll=8)
            pltpu.sync_copy(bufq, out_hbm.at[pl.ds(base + co, chunk)])
            return carry

        lax.fori_loop(0, n_chunks, chunk_body, 0)

    return sc_kernel(ids_flat, table)


def kernel(ids, table):
    n = ids.shape[0] * ids.shape[1]
    out = _lookup_add(ids.reshape(-1), table, n)
    return out.reshape(ids.shape + (_EMBED,))


# speedup vs baseline: 14.8945x; 14.8945x over previous
import functools

import jax
import jax.numpy as jnp
import numpy as np
from jax import lax
from jax.experimental import pallas as pl
from jax.experimental.pallas import tpu as pltpu
from jax.experimental.pallas import tpu_sc as plsc

_NUM_ROWS = 1000000
_EMBED = 16
_Q_MASK = np.int32(-65536)
_R_MASK = np.int32(65535)

_NC = 2
_NS = 16
_NW = _NC * _NS
_L = 16

_INV_ROWS = np.float32(1.0 / _NUM_ROWS)


def _mod_rows(x):
    q = (x.astype(jnp.float32) * _INV_ROWS).astype(jnp.int32)
    r = x - q * _NUM_ROWS
    r = jnp.where(r < 0, r + _NUM_ROWS, r)
    r = jnp.where(r >= _NUM_ROWS, r - _NUM_ROWS, r)
    return r


@functools.partial(jax.jit, static_argnames=("n",))
def _lookup_add(ids_flat, table, n):
    per_w = n // _NW
    chunk = 1024
    n_chunks = per_w // chunk
    n_vecs = per_w // _L

    mesh = plsc.VectorSubcoreMesh(core_axis_name="c", subcore_axis_name="s")

    @functools.partial(
        pl.kernel,
        out_type=jax.ShapeDtypeStruct((n, _EMBED), jnp.float32),
        mesh=mesh,
        scratch_types=[
            pltpu.VMEM((per_w,), jnp.int32),
            pltpu.VMEM((per_w,), jnp.int32),
            pltpu.VMEM((per_w,), jnp.int32),
            pltpu.VMEM((chunk, _EMBED), jnp.float32),
            pltpu.VMEM((chunk, _EMBED), jnp.float32),
            pltpu.SemaphoreType.DMA,
            pltpu.SemaphoreType.DMA,
        ],
        compiler_params=pltpu.CompilerParams(use_tc_tiling_on_sc=False),
    )
    def sc_kernel(ids_hbm, table_hbm, out_hbm,
                  ids_v, idxq_v, idxr_v, bufq, bufr, semq, semr):
        wid = lax.axis_index("s") * _NC + lax.axis_index("c")
        base = wid * per_w

        pltpu.sync_copy(ids_hbm.at[pl.ds(base, per_w)], ids_v)

        def idx_body(i, carry):
            o = i * _L
            x = ids_v[pl.ds(o, _L)]
            idxq_v[pl.ds(o, _L)] = _mod_rows(x & _Q_MASK)
            idxr_v[pl.ds(o, _L)] = x & _R_MASK
            return carry

        lax.fori_loop(0, n_vecs, idx_body, 0, unroll=4)

        def chunk_body(c, carry):
            co = c * chunk
            cpq = pltpu.async_copy(
                table_hbm.at[idxq_v.at[pl.ds(co, chunk)]], bufq, semq)
            cpr = pltpu.async_copy(
                table_hbm.at[idxr_v.at[pl.ds(co, chunk)]], bufr, semr)
            cpq.wait()
            cpr.wait()

            def add_body(i, carry2):
                bufq[i] = bufq[i] + bufr[i]
                return carry2

            lax.fori_loop(0, chunk, add_body, 0, unroll=8)
            pltpu.sync_copy(bufq, out_hbm.at[pl.ds(base + co, chunk)])
            return carry

        lax.fori_loop(0, n_chunks, chunk_body, 0)

    return sc_kernel(ids_flat, table)


def kernel(ids, table):
    n = ids.shape[0] * ids.shape[1]
    out = _lookup_add(ids.reshape(-1), table, n)
    return out.reshape(ids.shape + (_EMBED,))


# speedup vs baseline: 24.1514x; 1.6215x over previous
import functools

import jax
import jax.numpy as jnp
import numpy as np
from jax import lax
from jax.experimental import pallas as pl
from jax.experimental.pallas import tpu as pltpu
from jax.experimental.pallas import tpu_sc as plsc

_NUM_ROWS = 1000000
_EMBED = 16
_R_MASK = np.int32(65535)
_R_ROWS = 65536
_Q_ROWS = 32768

_NC = 2
_NS = 16
_NW = _NC * _NS
_L = 16

_CHUNK = 832
_NBUF = 2


@functools.partial(jax.jit, static_argnames=("n",))
def _lookup_add(ids_flat, sub_table, n):
    per_w = n // _NW
    n_chunks = per_w // _CHUNK
    vecs_per_chunk = _CHUNK // _L
    assert per_w % _CHUNK == 0 and _CHUNK % _L == 0

    mesh = plsc.VectorSubcoreMesh(core_axis_name="c", subcore_axis_name="s")

    @functools.partial(
        pl.kernel,
        out_type=jax.ShapeDtypeStruct((n, _EMBED), jnp.float32),
        mesh=mesh,
        scratch_types=[
            pltpu.VMEM((per_w,), jnp.int32),
            pltpu.VMEM((per_w,), jnp.int32),
            [pltpu.VMEM((_CHUNK, _EMBED), jnp.float32) for _ in range(_NBUF)],
            [pltpu.VMEM((_CHUNK, _EMBED), jnp.float32) for _ in range(_NBUF)],
            [pltpu.VMEM((_CHUNK, _EMBED), jnp.float32) for _ in range(_NBUF)],
            [pltpu.SemaphoreType.DMA for _ in range(_NBUF)],
            [pltpu.SemaphoreType.DMA for _ in range(_NBUF)],
        ],
        compiler_params=pltpu.CompilerParams(use_tc_tiling_on_sc=False),
    )
    def sc_kernel(ids_hbm, sub_hbm, out_hbm,
                  idxq_v, idxr_v, bq, br, bo, sg, sw):
        wid = lax.axis_index("s") * _NC + lax.axis_index("c")
        base = wid * per_w

        pltpu.sync_copy(ids_hbm.at[pl.ds(base, per_w)], idxq_v)

        def compute_idx(cc):
            def body(j, carry):
                o = cc * _CHUNK + j * _L
                x = idxq_v[pl.ds(o, _L)]
                idxr_v[pl.ds(o, _L)] = x & _R_MASK
                idxq_v[pl.ds(o, _L)] = (
                    lax.shift_right_logical(x, 16) + _R_ROWS)
                return carry
            lax.fori_loop(0, vecs_per_chunk, body, 0, unroll=4)

        def issue(cc):
            p = cc % _NBUF
            co = cc * _CHUNK
            dq = pltpu.async_copy(
                sub_hbm.at[idxq_v.at[pl.ds(co, _CHUNK)]], bq[p], sg[p])
            dr = pltpu.async_copy(
                sub_hbm.at[idxr_v.at[pl.ds(co, _CHUNK)]], br[p], sg[p])
            return dq, dr

        gth = {}
        for cc in range(_NBUF):
            compute_idx(cc)
            gth[cc] = issue(cc)

        wb = {}
        for cc in range(n_chunks):
            p = cc % _NBUF
            nxt = cc + _NBUF
            if nxt < n_chunks:
                compute_idx(nxt)
            dq, dr = gth.pop(cc)
            dq.wait()
            dr.wait()
            if cc >= _NBUF:
                wb.pop(cc - _NBUF).wait()

            def add_body(i, carry, p=p):
                bo[p][i] = bq[p][i] + br[p][i]
                return carry
            lax.fori_loop(0, _CHUNK, add_body, 0, unroll=8)

            wb[cc] = pltpu.async_copy(
                bo[p], out_hbm.at[pl.ds(base + cc * _CHUNK, _CHUNK)], sw[p])
            if nxt < n_chunks:
                gth[nxt] = issue(nxt)

        for cc in sorted(wb):
            wb[cc].wait()

    return sc_kernel(ids_flat, sub_table)


def kernel(ids, table):
    h = jnp.arange(_Q_ROWS, dtype=jnp.int32)
    qrows = (h * np.int32(65536)) % np.int32(_NUM_ROWS)
    sub_table = jnp.concatenate([table[:_R_ROWS], table[qrows]], axis=0)
    n = ids.shape[0] * ids.shape[1]
    out = _lookup_add(ids.reshape(-1), sub_table, n)
    return out.reshape(ids.shape + (_EMBED,))


# speedup vs baseline: 55.1294x; 2.2827x over previous
import functools

import jax
import jax.numpy as jnp
import numpy as np
from jax import lax
from jax.experimental import pallas as pl
from jax.experimental.pallas import tpu as pltpu
from jax.experimental.pallas import tpu_sc as plsc

_NUM_ROWS = 1000000
_EMBED = 16
_R_MASK = np.int32(65535)
_R_ROWS = 65536
_Q_ROWS = 32768

_NC = 2
_NS = 16
_NW = _NC * _NS
_L = 16

_CHUNK = 512
_NBUF = 2


@functools.partial(jax.jit, static_argnames=("batch", "fields"))
def _lookup_add(ids_flat, sub_table, batch, fields):
    n = batch * fields
    per_w = n // _NW
    n_chunks = per_w // _CHUNK
    vecs_per_chunk = _CHUNK // _L
    chunks_per_field = batch // _CHUNK
    assert per_w % _CHUNK == 0 and batch % _CHUNK == 0
    assert chunks_per_field & (chunks_per_field - 1) == 0
    cpf_shift = chunks_per_field.bit_length() - 1

    mesh = plsc.VectorSubcoreMesh(core_axis_name="c", subcore_axis_name="s")

    @functools.partial(
        pl.kernel,
        out_type=jax.ShapeDtypeStruct((fields, _EMBED, batch), jnp.float32),
        mesh=mesh,
        scratch_types=[
            pltpu.VMEM((per_w,), jnp.int32),
            pltpu.VMEM((per_w,), jnp.int32),
            [pltpu.VMEM((_CHUNK, _EMBED), jnp.float32) for _ in range(_NBUF)],
            [pltpu.VMEM((_CHUNK, _EMBED), jnp.float32) for _ in range(_NBUF)],
            [pltpu.VMEM((_EMBED, _CHUNK), jnp.float32) for _ in range(_NBUF)],
            [pltpu.SemaphoreType.DMA for _ in range(_NBUF)],
            [pltpu.SemaphoreType.DMA for _ in range(_NBUF)],
        ],
        compiler_params=pltpu.CompilerParams(
            use_tc_tiling_on_sc=False, needs_layout_passes=False),
    )
    def sc_kernel(ids_hbm, sub_hbm, out_hbm,
                  idxq_v, idxr_v, bq, br, bo, sg, sw):
        wid = lax.axis_index("s") * _NC + lax.axis_index("c")
        base = wid * per_w

        pltpu.sync_copy(ids_hbm.at[pl.ds(base, per_w)], idxq_v)

        def compute_idx(cc):
            def body(j, carry):
                o = cc * _CHUNK + j * _L
                x = idxq_v[pl.ds(o, _L)]
                idxr_v[pl.ds(o, _L)] = x & _R_MASK
                idxq_v[pl.ds(o, _L)] = (
                    lax.shift_right_logical(x, 16) + _R_ROWS)
                return carry
            lax.fori_loop(0, vecs_per_chunk, body, 0, unroll=4)

        def issue(cc):
            p = cc % _NBUF
            co = cc * _CHUNK
            dq = pltpu.async_copy(
                sub_hbm.at[idxq_v.at[pl.ds(co, _CHUNK)]], bq[p], sg[p])
            dr = pltpu.async_copy(
                sub_hbm.at[idxr_v.at[pl.ds(co, _CHUNK)]], br[p], sg[p])
            return dq, dr

        lane = lax.iota(jnp.int32, _L)

        gth = {}
        for cc in range(_NBUF):
            compute_idx(cc)
            gth[cc] = issue(cc)

        wb = {}
        for cc in range(n_chunks):
            p = cc % _NBUF
            nxt = cc + _NBUF
            if nxt < n_chunks:
                compute_idx(nxt)
            dq, dr = gth.pop(cc)
            dq.wait()
            dr.wait()
            if cc >= _NBUF:
                wb.pop(cc - _NBUF).wait()

            def add_body(i, carry, p=p):
                v = bq[p][i] + br[p][i]
                plsc.store_scatter(bo[p], (lane, jnp.full((_L,), i,
                                                          jnp.int32)), v)
                return carry
            lax.fori_loop(0, _CHUNK, add_body, 0, unroll=8)

            g = wid * n_chunks + cc
            f = lax.shift_right_logical(g, cpf_shift)
            b0 = (g & (chunks_per_field - 1)) * _CHUNK
            wb[cc] = pltpu.async_copy(
                bo[p], out_hbm.at[f].at[:, pl.ds(b0, _CHUNK)], sw[p])
            if nxt < n_chunks:
                gth[nxt] = issue(nxt)

        for cc in sorted(wb):
            wb[cc].wait()

    return sc_kernel(ids_flat, sub_table)


def kernel(ids, table):
    batch, fields = ids.shape
    h = jnp.arange(_Q_ROWS, dtype=jnp.int32)
    qrows = (h * np.int32(65536)) % np.int32(_NUM_ROWS)
    sub_table = jnp.concatenate([table[:_R_ROWS], table[qrows]], axis=0)
    out = _lookup_add(ids.T.reshape(-1), sub_table, batch, fields)
    return jnp.transpose(out, (2, 0, 1))


# speedup vs baseline: 55.3557x; 1.0041x over previous
import functools

import jax
import jax.numpy as jnp
import numpy as np
from jax import lax
from jax.experimental import pallas as pl
from jax.experimental.pallas import tpu as pltpu
from jax.experimental.pallas import tpu_sc as plsc

_NUM_ROWS = 1000000
_EMBED = 16
_R_MASK = np.int32(65535)
_R_ROWS = 65536
_Q_ROWS = 32768

_NC = 2
_NS = 16
_NW = _NC * _NS
_L = 16

_CHUNK = 512
_NBUF = 2


@functools.partial(jax.jit, static_argnames=("batch", "fields"))
def _lookup_add(ids_flat, sub_table, batch, fields):
    n = batch * fields
    per_w = n // _NW
    n_chunks = per_w // _CHUNK
    vecs_per_chunk = _CHUNK // _L
    chunks_per_field = batch // _CHUNK
    assert per_w % _CHUNK == 0 and batch % _CHUNK == 0
    assert chunks_per_field & (chunks_per_field - 1) == 0
    cpf_shift = chunks_per_field.bit_length() - 1

    mesh = plsc.VectorSubcoreMesh(core_axis_name="c", subcore_axis_name="s")

    @functools.partial(
        pl.kernel,
        out_type=jax.ShapeDtypeStruct((fields, _EMBED, batch), jnp.float32),
        mesh=mesh,
        scratch_types=[
            pltpu.VMEM((per_w,), jnp.int32),
            pltpu.VMEM((per_w,), jnp.int32),
            [pltpu.VMEM((_CHUNK, _EMBED), jnp.float32) for _ in range(_NBUF)],
            [pltpu.VMEM((_CHUNK, _EMBED), jnp.float32) for _ in range(_NBUF)],
            [pltpu.VMEM((_EMBED, _CHUNK), jnp.float32) for _ in range(_NBUF)],
            [pltpu.SemaphoreType.DMA for _ in range(_NBUF)],
            [pltpu.SemaphoreType.DMA for _ in range(_NBUF)],
        ],
        compiler_params=pltpu.CompilerParams(
            use_tc_tiling_on_sc=False, needs_layout_passes=False),
    )
    def sc_kernel(ids_hbm, sub_hbm, out_hbm,
                  idxq_v, idxr_v, bq, br, bo, sg, sw):
        wid = lax.axis_index("s") * _NC + lax.axis_index("c")
        base = wid * per_w

        pltpu.sync_copy(ids_hbm.at[pl.ds(base, per_w)], idxq_v)

        def compute_idx(cc):
            def body(j, carry):
                o = cc * _CHUNK + j * _L
                x = idxq_v[pl.ds(o, _L)]
                idxr_v[pl.ds(o, _L)] = x & _R_MASK
                idxq_v[pl.ds(o, _L)] = (
                    lax.shift_right_logical(x, 16) + _R_ROWS)
                return carry
            lax.fori_loop(0, vecs_per_chunk, body, 0, unroll=4)

        def issue(cc):
            p = cc % _NBUF
            co = cc * _CHUNK
            dq = pltpu.async_copy(
                sub_hbm.at[idxq_v.at[pl.ds(co, _CHUNK)]], bq[p], sg[p])
            dr = pltpu.async_copy(
                sub_hbm.at[idxr_v.at[pl.ds(co, _CHUNK)]], br[p], sg[p])
            return dq, dr

        lane = lax.iota(jnp.int32, _L)
        zeros = lane * 0

        gth = {}
        for cc in range(_NBUF):
            compute_idx(cc)
            gth[cc] = issue(cc)

        wb = {}
        for cc in range(n_chunks):
            p = cc % _NBUF
            nxt = cc + _NBUF
            if nxt < n_chunks:
                compute_idx(nxt)
            dq, dr = gth.pop(cc)
            dq.wait()
            dr.wait()
            if cc >= _NBUF:
                wb.pop(cc - _NBUF).wait()

            def add_body(i, ivec, p=p):
                v = bq[p][i] + br[p][i]
                plsc.store_scatter(bo[p], (lane, ivec), v)
                return ivec + 1
            lax.fori_loop(0, _CHUNK, add_body, zeros, unroll=8)

            g = wid * n_chunks + cc
            f = lax.shift_right_logical(g, cpf_shift)
            b0 = (g & (chunks_per_field - 1)) * _CHUNK
            wb[cc] = pltpu.async_copy(
                bo[p], out_hbm.at[f].at[:, pl.ds(b0, _CHUNK)], sw[p])
            if nxt < n_chunks:
                gth[nxt] = issue(nxt)

        for cc in sorted(wb):
            wb[cc].wait()

    return sc_kernel(ids_flat, sub_table)


def kernel(ids, table):
    batch, fields = ids.shape
    h = jnp.arange(_Q_ROWS, dtype=jnp.int32)
    qrows = (h * np.int32(65536)) % np.int32(_NUM_ROWS)
    sub_table = jnp.concatenate([table[:_R_ROWS], table[qrows]], axis=0)
    out = _lookup_add(ids.T.reshape(-1), sub_table, batch, fields)
    return jnp.transpose(out, (2, 0, 1))
